# Initial kernel scaffold; baseline (speedup 1.0000x reference)
import jax
import jax.numpy as jnp
from jax.experimental import pallas as pl


def kernel(voxel_feat, coors):
    raise NotImplementedError("write your pallas kernel here")


# speedup vs baseline: 3.9101x; 3.9101x over previous
import functools

import jax
import jax.numpy as jnp
from jax import lax
from jax.experimental import pallas as pl
from jax.experimental.pallas import tpu as pltpu
from jax.experimental.pallas import tpu_sc as plsc

N = 32768
D = 128
W = 25600
CH = 2048
NCH = N // CH
VR = CH // 16

MW_PER_SAMPLE = 1600
MWY = 40


def _iota16():
    return lax.iota(jnp.int32, 16)


def _lvl_target(cntw):
    return jnp.where(cntw < 30, 30, jnp.where(cntw < 60, 60, 100))


def _body(coors_hbm, feat_hbm,
          feat_out, coors_out, bwi0_out, bwi1_out, l0_out, l1_out,
          f2w0_out, f2w1_out,
          t_c4, t_bwi0, t_bwi1, t_g1, t_g2, t_g3, t_g4, t_hist,
          t_idx, t_feat,
          s_coors, s_bwi0, s_bwi1, s_bwi1p, s_bwi0f, s_inner, s_inner1,
          s_innerF0, s_innerF1, s_keep, s_pos, s_perm1, s_P,
          s_cnt0, s_cnt1, s_base0, s_base1):
    sid = lax.axis_index("s")
    cid = lax.axis_index("c")

    def zero_hist():
        z = jnp.zeros((16,), jnp.int32)

        def zb(j, _):
            t_hist[pl.ds(j * 16, 16)] = z
            return 0

        lax.fori_loop(0, W // 16, zb, 0)

    def rank_chunk(c):
        def rb(i, _):
            w = t_bwi0[pl.ds(i * 16, 16)]
            cnt, last = plsc.scan_count(w)
            old = plsc.load_gather(t_hist, [w])
            t_g1[pl.ds(i * 16, 16)] = old + cnt - 1
            plsc.store_scatter(t_hist, [w], old + cnt, mask=last)
            return 0

        lax.fori_loop(0, VR, rb, 0)

    def rank_pass(load_chunk, s_inner_dst):
        zero_hist()

        def cb(c, _):
            load_chunk(c)
            rank_chunk(c)
            pltpu.sync_copy(t_g1, s_inner_dst.at[pl.ds(c * CH, CH)])
            return 0

        lax.fori_loop(0, NCH, cb, 0)

    def keep_pos_pass(s_bwi_src, s_inner_src):
        def cb(c, nknd):
            nk, nd = nknd
            pltpu.sync_copy(s_bwi_src.at[pl.ds(c * CH, CH)], t_bwi0)
            pltpu.sync_copy(s_inner_src.at[pl.ds(c * CH, CH)], t_g2)

            def ib(i, nknd):
                nk, nd = nknd
                w = t_bwi0[pl.ds(i * 16, 16)]
                inner = t_g2[pl.ds(i * 16, 16)]
                cntw = plsc.load_gather(t_hist, [w])
                keep = inner < _lvl_target(cntw)
                k = jnp.where(keep, 1, 0)
                cs = plsc.cumsum(k)
                kept_rank = nk + cs - k
                drop_rank = nd + (_iota16() + 1 - cs) - (1 - k)
                t_g3[pl.ds(i * 16, 16)] = jnp.where(keep, kept_rank, drop_rank)
                t_g4[pl.ds(i * 16, 16)] = k
                s = jnp.sum(k)
                return (nk + s, nd + 16 - s)

            nk, nd = lax.fori_loop(0, VR, ib, (nk, nd))
            pltpu.sync_copy(t_g3, s_pos.at[pl.ds(c * CH, CH)])
            pltpu.sync_copy(t_g4, s_keep.at[pl.ds(c * CH, CH)])
            return (nk, nd)

        nk, _ = lax.fori_loop(0, NCH, cb, (jnp.int32(0), jnp.int32(0)))
        return nk

    def scatter_perm(nkeep, fill_vals, dsts):
        def cb(c, _):
            pltpu.sync_copy(s_pos.at[pl.ds(c * CH, CH)], t_g1)
            pltpu.sync_copy(s_keep.at[pl.ds(c * CH, CH)], t_g2)

            def ib(i, _):
                posp = t_g1[pl.ds(i * 16, 16)]
                k = t_g2[pl.ds(i * 16, 16)]
                t_g3[pl.ds(i * 16, 16)] = posp + jnp.where(k > 0, 0, nkeep)
                return 0

            lax.fori_loop(0, VR, ib, 0)
            for fill, (vref, dst) in zip(fill_vals, dsts):
                fill(c, vref)
                pltpu.sync_copy(vref, dst.at[t_g3])
            return 0

        lax.fori_loop(0, NCH, cb, 0)

    def base_table(s_cnt_src, s_base_dst):
        def cb(c, runs):
            pltpu.sync_copy(s_cnt_src.at[pl.ds(c * 1600, 1600)],
                            t_g1.at[pl.ds(0, 1600)])

            def ib(j, runs):
                r0, r1, r2 = runs
                cnt = t_g1[pl.ds(j * 16, 16)]
                m0 = (cnt > 0) & (cnt < 30)
                m1 = (cnt >= 30) & (cnt < 60)
                m2 = cnt >= 60
                i0 = jnp.where(m0, 1, 0)
                i1 = jnp.where(m1, 1, 0)
                i2 = jnp.where(m2, 1, 0)
                c0 = plsc.cumsum(i0)
                c1 = plsc.cumsum(i1)
                c2 = plsc.cumsum(i2)
                base = jnp.where(
                    m0, (r0 + c0 - i0) * 30,
                    jnp.where(m1, (r1 + c1 - i1) * 60, (r2 + c2 - i2) * 100))
                t_g2[pl.ds(j * 16, 16)] = base
                return (r0 + jnp.sum(i0), r1 + jnp.sum(i1), r2 + jnp.sum(i2))

            runs = lax.fori_loop(0, 100, ib, runs)
            pltpu.sync_copy(t_g2.at[pl.ds(0, 1600)],
                            s_base_dst.at[pl.ds(c * 1600, 1600)])
            return runs

        lax.fori_loop(0, NCH, cb,
                      (jnp.int32(0), jnp.int32(0), jnp.int32(0)))

    @pl.when(sid == 0)
    def _pipeline():
        def a_chunk(c, _):
            pltpu.sync_copy(coors_hbm.at[pl.ds(c * CH * 4, CH * 4)], t_c4)
            pltpu.sync_copy(t_c4, s_coors.at[pl.ds(c * CH * 4, CH * 4)])

            def ib(i, _):
                lanes = _iota16() * 4 + i * 64
                b = plsc.load_gather(t_c4, [lanes])
                y = plsc.load_gather(t_c4, [lanes + 2])
                x = plsc.load_gather(t_c4, [lanes + 3])
                bwi0 = b * MW_PER_SAMPLE + (x // 12) * MWY + (y // 12)
                bwi1 = (b * MW_PER_SAMPLE + ((x + 6) // 12) * MWY
                        + ((y + 6) // 12))
                t_bwi0[pl.ds(i * 16, 16)] = bwi0
                t_bwi1[pl.ds(i * 16, 16)] = bwi1
                return 0

            lax.fori_loop(0, VR, ib, 0)
            pltpu.sync_copy(t_bwi0, s_bwi0.at[pl.ds(c * CH, CH)])
            pltpu.sync_copy(t_bwi1, s_bwi1.at[pl.ds(c * CH, CH)])
            return 0

        lax.fori_loop(0, NCH, a_chunk, 0)

        def load0(c):
            pltpu.sync_copy(s_bwi0.at[pl.ds(c * CH, CH)], t_bwi0)

        rank_pass(load0, s_inner)
        pltpu.sync_copy(t_hist, s_cnt0)

        nk0 = keep_pos_pass(s_bwi0, s_inner)

        def fill_iota(c, vref):
            def ib(i, _):
                vref[pl.ds(i * 16, 16)] = _iota16() + (c * CH + i * 16)
                return 0
            lax.fori_loop(0, VR, ib, 0)

        scatter_perm(nk0, [fill_iota], [(t_g4, s_perm1)])

        def load1(c):
            pltpu.sync_copy(s_perm1.at[pl.ds(c * CH, CH)], t_g4)
            pltpu.sync_copy(s_bwi1.at[t_g4], t_bwi0)
            pltpu.sync_copy(t_bwi0, s_bwi1p.at[pl.ds(c * CH, CH)])

        rank_pass(load1, s_inner1)
        pltpu.sync_copy(t_hist, s_cnt1)

        nk1 = keep_pos_pass(s_bwi1p, s_inner1)

        def fill_perm1(c, vref):
            pltpu.sync_copy(s_perm1.at[pl.ds(c * CH, CH)], vref)

        def fill_inner1(c, vref):
            pltpu.sync_copy(s_inner1.at[pl.ds(c * CH, CH)], vref)

        scatter_perm(nk1, [fill_perm1, fill_inner1],
                     [(t_g4, s_P), (t_bwi1, s_innerF1)])

        def loadf(c):
            pltpu.sync_copy(s_P.at[pl.ds(c * CH, CH)], t_g4)
            pltpu.sync_copy(s_bwi0.at[t_g4], t_bwi0)
            pltpu.sync_copy(t_bwi0, s_bwi0f.at[pl.ds(c * CH, CH)])

        rank_pass(loadf, s_innerF0)

        base_table(s_cnt0, s_base0)
        base_table(s_cnt1, s_base1)

    plsc.subcore_barrier()

    @pl.when(cid == 0)
    def _int_outputs():
        base = sid * CH
        sl = pl.ds(base, CH)
        pltpu.sync_copy(s_P.at[sl], t_g4)
        pltpu.sync_copy(s_bwi0f.at[sl], t_bwi0)
        pltpu.sync_copy(s_bwi1.at[t_g4], t_bwi1)
        pltpu.sync_copy(t_bwi0, bwi0_out.at[sl])
        pltpu.sync_copy(t_bwi1, bwi1_out.at[sl])
        for fld in range(4):
            def gib(i, _):
                t_g1[pl.ds(i * 16, 16)] = t_g4[pl.ds(i * 16, 16)] * 4 + fld
                return 0

            lax.fori_loop(0, VR, gib, 0)
            pltpu.sync_copy(s_coors.at[t_g1], t_g2)

            def sib(i, _):
                v = t_g2[pl.ds(i * 16, 16)]
                plsc.store_scatter(t_c4, [_iota16() * 4 + i * 64 + fld], v)
                return 0

            lax.fori_loop(0, VR, sib, 0)
        pltpu.sync_copy(t_c4, coors_out.at[pl.ds(base * 4, CH * 4)])

        def lvl_out(s_cnt_src, bwi_ref, dst):
            pltpu.sync_copy(s_cnt_src.at[bwi_ref], t_g1)

            def ib(i, _):
                cnt = t_g1[pl.ds(i * 16, 16)]
                t_g2[pl.ds(i * 16, 16)] = jnp.where(
                    cnt < 30, 0, jnp.where(cnt < 60, 1, 2))
                return 0

            lax.fori_loop(0, VR, ib, 0)
            pltpu.sync_copy(t_g2, dst.at[sl])

        lvl_out(s_cnt0, t_bwi0, l0_out)
        lvl_out(s_cnt1, t_bwi1, l1_out)

        def f2w_out(s_base_src, bwi_ref, s_innerF_src, dst):
            pltpu.sync_copy(s_base_src.at[bwi_ref], t_g1)
            pltpu.sync_copy(s_innerF_src.at[sl], t_g2)

            def ib(i, _):
                t_g3[pl.ds(i * 16, 16)] = (t_g1[pl.ds(i * 16, 16)]
                                           + t_g2[pl.ds(i * 16, 16)])
                return 0

            lax.fori_loop(0, VR, ib, 0)
            pltpu.sync_copy(t_g3, dst.at[sl])

        f2w_out(s_base0, t_bwi0, s_innerF0, f2w0_out)
        f2w_out(s_base1, t_bwi1, s_innerF1, f2w1_out)

    wid = cid * 16 + sid
    rows_per_w = N // 32
    pltpu.sync_copy(s_P.at[pl.ds(wid * rows_per_w, rows_per_w)], t_idx)

    def fb(p, _):
        pltpu.sync_copy(feat_hbm.at[t_idx.at[pl.ds(p * 128, 128)]], t_feat)
        pltpu.sync_copy(t_feat,
                        feat_out.at[pl.ds(wid * rows_per_w + p * 128, 128)])
        return 0

    lax.fori_loop(0, rows_per_w // 128, fb, 0)


@functools.partial(jax.jit, static_argnames=())
def kernel(voxel_feat, coors):
    coors = coors.astype(jnp.int32)
    mesh = plsc.VectorSubcoreMesh(core_axis_name="c", subcore_axis_name="s",
                                  num_cores=2, num_subcores=16)
    f = pl.kernel(
        _body,
        out_type=(
            jax.ShapeDtypeStruct((N, D), jnp.float32),
            jax.ShapeDtypeStruct((N * 4,), jnp.int32),
            jax.ShapeDtypeStruct((N,), jnp.int32),
            jax.ShapeDtypeStruct((N,), jnp.int32),
            jax.ShapeDtypeStruct((N,), jnp.int32),
            jax.ShapeDtypeStruct((N,), jnp.int32),
            jax.ShapeDtypeStruct((N,), jnp.int32),
            jax.ShapeDtypeStruct((N,), jnp.int32),
        ),
        mesh=mesh,
        scratch_types=[
            pltpu.VMEM((CH * 4,), jnp.int32),
            pltpu.VMEM((CH,), jnp.int32),
            pltpu.VMEM((CH,), jnp.int32),
            pltpu.VMEM((CH,), jnp.int32),
            pltpu.VMEM((CH,), jnp.int32),
            pltpu.VMEM((CH,), jnp.int32),
            pltpu.VMEM((CH,), jnp.int32),
            pltpu.VMEM((W,), jnp.int32),
            pltpu.VMEM((N // 32,), jnp.int32),
            pltpu.VMEM((128, D), jnp.float32),
            pltpu.VMEM_SHARED((N * 4,), jnp.int32),
            pltpu.VMEM_SHARED((N,), jnp.int32),
            pltpu.VMEM_SHARED((N,), jnp.int32),
            pltpu.VMEM_SHARED((N,), jnp.int32),
            pltpu.VMEM_SHARED((N,), jnp.int32),
            pltpu.VMEM_SHARED((N,), jnp.int32),
            pltpu.VMEM_SHARED((N,), jnp.int32),
            pltpu.VMEM_SHARED((N,), jnp.int32),
            pltpu.VMEM_SHARED((N,), jnp.int32),
            pltpu.VMEM_SHARED((N,), jnp.int32),
            pltpu.VMEM_SHARED((N,), jnp.int32),
            pltpu.VMEM_SHARED((N,), jnp.int32),
            pltpu.VMEM_SHARED((N,), jnp.int32),
            pltpu.VMEM_SHARED((W,), jnp.int32),
            pltpu.VMEM_SHARED((W,), jnp.int32),
            pltpu.VMEM_SHARED((W,), jnp.int32),
            pltpu.VMEM_SHARED((W,), jnp.int32),
        ],
        compiler_params=pltpu.CompilerParams(needs_layout_passes=False),
    )
    out = f(jnp.reshape(coors, (N * 4,)), voxel_feat)
    (feat_f, coors_flat, bwi0_f, bwi1_f, l0_f, l1_f, f2w0_f, f2w1_f) = out
    return (feat_f, jnp.reshape(coors_flat, (N, 4)), bwi0_f, bwi1_f,
            l0_f, l1_f, f2w0_f, f2w1_f)


# speedup vs baseline: 16.3675x; 4.1859x over previous
import functools

import jax
import jax.numpy as jnp
from jax import lax
from jax.experimental import pallas as pl
from jax.experimental.pallas import tpu as pltpu
from jax.experimental.pallas import tpu_sc as plsc

N = 32768
D = 128
W = 25600
CH = 2048
NCH = N // CH
VR = CH // 16
WSL = W // 16

MW_PER_SAMPLE = 1600
MWY = 40


def _iota16():
    return lax.iota(jnp.int32, 16)


def _lvl_target(cntw):
    return jnp.where(cntw < 30, 30, jnp.where(cntw < 60, 60, 100))


def _body(coors_hbm, feat_hbm,
          feat_out, coors_out, bwi0_out, bwi1_out, l0_out, l1_out,
          f2w0_out, f2w1_out,
          t_c4, t_bwi0, t_bwi1, t_g1, t_g2, t_g3, t_g4, t_g5, t_hist,
          t_small, t_idx, t_feat,
          s_coors, s_bwi0, s_bwi1, s_perm1, s_P, s_innerF1,
          s_hist, s_cnt0, s_cnt1, s_base0, s_base1, s_small):
    sid = lax.axis_index("s")
    cid = lax.axis_index("c")
    base = sid * CH
    sl = pl.ds(base, CH)

    def zero_hist():
        z = jnp.zeros((16,), jnp.int32)

        def zb(j, _):
            t_hist[pl.ds(j * 16, 16)] = z
            return 0

        lax.fori_loop(0, W // 16, zb, 0)

    def rank_local():
        def rb(i, _):
            w = t_bwi0[pl.ds(i * 16, 16)]
            cnt, last = plsc.scan_count(w)
            old = plsc.load_gather(t_hist, [w])
            t_g1[pl.ds(i * 16, 16)] = old + cnt - 1
            plsc.store_scatter(t_hist, [w], old + cnt, mask=last)
            return 0

        lax.fori_loop(0, VR, rb, 0)

    def combine_hist(s_cnt_dst):
        pltpu.sync_copy(t_hist, s_hist.at[pl.ds(sid * W, W)])
        plsc.subcore_barrier()
        for r in range(16):
            pltpu.sync_copy(s_hist.at[pl.ds(r * W + sid * WSL, WSL)],
                            t_hist.at[pl.ds(r * WSL, WSL)])

        def jb(j, _):
            acc = jnp.zeros((16,), jnp.int32)
            for r in range(16):
                o = r * WSL + j * 16
                v = t_hist[pl.ds(o, 16)]
                t_hist[pl.ds(o, 16)] = acc
                acc = acc + v
            t_g2[pl.ds(j * 16, 16)] = acc
            return 0

        lax.fori_loop(0, WSL // 16, jb, 0)
        for r in range(16):
            pltpu.sync_copy(t_hist.at[pl.ds(r * WSL, WSL)],
                            s_hist.at[pl.ds(r * W + sid * WSL, WSL)])
        pltpu.sync_copy(t_g2.at[pl.ds(0, WSL)],
                        s_cnt_dst.at[pl.ds(sid * WSL, WSL)])
        plsc.subcore_barrier()
        pltpu.sync_copy(s_hist.at[pl.ds(sid * W, W)], t_hist)

        def fb(i, _):
            w = t_bwi0[pl.ds(i * 16, 16)]
            t_g5[pl.ds(i * 16, 16)] = (t_g1[pl.ds(i * 16, 16)]
                                       + plsc.load_gather(t_hist, [w]))
            return 0

        lax.fori_loop(0, VR, fb, 0)

    def keep_scatter(s_cnt_src, fills):
        pltpu.sync_copy(s_cnt_src.at[t_bwi0], t_g2)

        def ib(i, nknd):
            nk, nd = nknd
            cntw = t_g2[pl.ds(i * 16, 16)]
            keep = t_g5[pl.ds(i * 16, 16)] < _lvl_target(cntw)
            k = jnp.where(keep, 1, 0)
            cs = plsc.cumsum(k)
            kept_rank = nk + cs - k
            drop_rank = nd + (_iota16() + 1 - cs) - (1 - k)
            t_g3[pl.ds(i * 16, 16)] = jnp.where(keep, kept_rank, drop_rank)
            t_g4[pl.ds(i * 16, 16)] = k
            s = jnp.sum(k)
            return (nk + s, nd + 16 - s)

        nk, nd = lax.fori_loop(0, VR, ib, (jnp.int32(0), jnp.int32(0)))
        io = _iota16()
        t_small[pl.ds(0, 16)] = jnp.where(io == 0, nk,
                                          jnp.where(io == 1, nd, 0))
        pltpu.sync_copy(t_small.at[pl.ds(0, 16)],
                        s_small.at[pl.ds(sid * 16, 16)])
        plsc.subcore_barrier()
        pltpu.sync_copy(s_small, t_small)
        nk_v = plsc.load_gather(t_small, [io * 16])
        nd_v = plsc.load_gather(t_small, [io * 16 + 1])
        before = io < sid
        k_off = jnp.sum(jnp.where(before, nk_v, 0))
        d_off = jnp.sum(jnp.where(before, nd_v, 0))
        nkeep = jnp.sum(nk_v)

        def pb(i, _):
            k = t_g4[pl.ds(i * 16, 16)]
            t_g3[pl.ds(i * 16, 16)] = (t_g3[pl.ds(i * 16, 16)]
                                       + jnp.where(k > 0, k_off,
                                                   nkeep + d_off))
            return 0

        lax.fori_loop(0, VR, pb, 0)
        for fill, vref, dst in fills:
            if fill is not None:
                fill(vref)
            pltpu.sync_copy(vref, dst.at[t_g3])

    def base_table(s_cnt_src, s_base_dst):
        csl = pl.ds(sid * WSL, WSL)
        pltpu.sync_copy(s_cnt_src.at[csl], t_g1.at[pl.ds(0, WSL)])

        def cb(j, runs):
            r0, r1, r2 = runs
            cnt = t_g1[pl.ds(j * 16, 16)]
            i0 = jnp.where((cnt > 0) & (cnt < 30), 1, 0)
            i1 = jnp.where((cnt >= 30) & (cnt < 60), 1, 0)
            i2 = jnp.where(cnt >= 60, 1, 0)
            return (r0 + jnp.sum(i0), r1 + jnp.sum(i1), r2 + jnp.sum(i2))

        l0c, l1c, l2c = lax.fori_loop(
            0, WSL // 16, cb, (jnp.int32(0), jnp.int32(0), jnp.int32(0)))
        io = _iota16()
        t_small[pl.ds(0, 16)] = jnp.where(
            io == 0, l0c, jnp.where(io == 1, l1c,
                                    jnp.where(io == 2, l2c, 0)))
        pltpu.sync_copy(t_small.at[pl.ds(0, 16)],
                        s_small.at[pl.ds(sid * 16, 16)])
        plsc.subcore_barrier()
        pltpu.sync_copy(s_small, t_small)
        c0v = plsc.load_gather(t_small, [io * 16])
        c1v = plsc.load_gather(t_small, [io * 16 + 1])
        c2v = plsc.load_gather(t_small, [io * 16 + 2])
        before = io < sid
        r0 = jnp.sum(jnp.where(before, c0v, 0))
        r1 = jnp.sum(jnp.where(before, c1v, 0))
        r2 = jnp.sum(jnp.where(before, c2v, 0))

        def bb(j, runs):
            r0, r1, r2 = runs
            cnt = t_g1[pl.ds(j * 16, 16)]
            m0 = (cnt > 0) & (cnt < 30)
            m1 = (cnt >= 30) & (cnt < 60)
            m2 = cnt >= 60
            i0 = jnp.where(m0, 1, 0)
            i1 = jnp.where(m1, 1, 0)
            i2 = jnp.where(m2, 1, 0)
            c0 = plsc.cumsum(i0)
            c1 = plsc.cumsum(i1)
            c2 = plsc.cumsum(i2)
            b = jnp.where(
                m0, (r0 + c0 - i0) * 30,
                jnp.where(m1, (r1 + c1 - i1) * 60, (r2 + c2 - i2) * 100))
            t_g2[pl.ds(j * 16, 16)] = b
            return (r0 + jnp.sum(i0), r1 + jnp.sum(i1), r2 + jnp.sum(i2))

        lax.fori_loop(0, WSL // 16, bb, (r0, r1, r2))
        pltpu.sync_copy(t_g2.at[pl.ds(0, WSL)], s_base_dst.at[csl])

    pltpu.sync_copy(coors_hbm.at[pl.ds(base * 4, CH * 4)], t_c4)
    pltpu.sync_copy(t_c4, s_coors.at[pl.ds(base * 4, CH * 4)])

    def a_ib(i, _):
        lanes = _iota16() * 4 + i * 64
        b = plsc.load_gather(t_c4, [lanes])
        y = plsc.load_gather(t_c4, [lanes + 2])
        x = plsc.load_gather(t_c4, [lanes + 3])
        t_bwi0[pl.ds(i * 16, 16)] = (b * MW_PER_SAMPLE
                                     + (x // 12) * MWY + (y // 12))
        t_bwi1[pl.ds(i * 16, 16)] = (b * MW_PER_SAMPLE
                                     + ((x + 6) // 12) * MWY
                                     + ((y + 6) // 12))
        return 0

    lax.fori_loop(0, VR, a_ib, 0)
    pltpu.sync_copy(t_bwi0, s_bwi0.at[sl])
    pltpu.sync_copy(t_bwi1, s_bwi1.at[sl])

    zero_hist()
    rank_local()
    combine_hist(s_cnt0)

    def fill_iota(vref):
        def ib(i, _):
            vref[pl.ds(i * 16, 16)] = _iota16() + (base + i * 16)
            return 0
        lax.fori_loop(0, VR, ib, 0)

    keep_scatter(s_cnt0, [(fill_iota, t_g1, s_perm1)])
    plsc.subcore_barrier()

    pltpu.sync_copy(s_perm1.at[sl], t_g4)
    pltpu.sync_copy(s_bwi1.at[t_g4], t_bwi0)
    zero_hist()
    rank_local()
    combine_hist(s_cnt1)

    def fill_perm1(vref):
        pltpu.sync_copy(s_perm1.at[sl], vref)

    keep_scatter(s_cnt1, [(fill_perm1, t_g1, s_P),
                          (None, t_g5, s_innerF1)])
    plsc.subcore_barrier()

    pltpu.sync_copy(s_P.at[sl], t_g4)
    pltpu.sync_copy(s_bwi0.at[t_g4], t_bwi0)
    zero_hist()
    rank_local()
    combine_hist(s_cnt0)

    base_table(s_cnt0, s_base0)
    plsc.subcore_barrier()
    base_table(s_cnt1, s_base1)
    plsc.subcore_barrier()

    @pl.when(cid == 0)
    def _int_outputs():
        pltpu.sync_copy(s_bwi1.at[t_g4], t_bwi1)
        pltpu.sync_copy(t_bwi0, bwi0_out.at[sl])
        pltpu.sync_copy(t_bwi1, bwi1_out.at[sl])
        for fld in range(4):
            def gib(i, _):
                t_g1[pl.ds(i * 16, 16)] = t_g4[pl.ds(i * 16, 16)] * 4 + fld
                return 0

            lax.fori_loop(0, VR, gib, 0)
            pltpu.sync_copy(s_coors.at[t_g1], t_g2)

            def sib(i, _):
                v = t_g2[pl.ds(i * 16, 16)]
                plsc.store_scatter(t_c4, [_iota16() * 4 + i * 64 + fld], v)
                return 0

            lax.fori_loop(0, VR, sib, 0)
        pltpu.sync_copy(t_c4, coors_out.at[pl.ds(base * 4, CH * 4)])

        def lvl_out(s_cnt_src, bwi_ref, dst):
            pltpu.sync_copy(s_cnt_src.at[bwi_ref], t_g1)

            def ib(i, _):
                cnt = t_g1[pl.ds(i * 16, 16)]
                t_g2[pl.ds(i * 16, 16)] = jnp.where(
                    cnt < 30, 0, jnp.where(cnt < 60, 1, 2))
                return 0

            lax.fori_loop(0, VR, ib, 0)
            pltpu.sync_copy(t_g2, dst.at[sl])

        lvl_out(s_cnt0, t_bwi0, l0_out)
        lvl_out(s_cnt1, t_bwi1, l1_out)

        pltpu.sync_copy(s_base0.at[t_bwi0], t_g1)

        def f0b(i, _):
            t_g3[pl.ds(i * 16, 16)] = (t_g1[pl.ds(i * 16, 16)]
                                       + t_g5[pl.ds(i * 16, 16)])
            return 0

        lax.fori_loop(0, VR, f0b, 0)
        pltpu.sync_copy(t_g3, f2w0_out.at[sl])

        pltpu.sync_copy(s_base1.at[t_bwi1], t_g1)
        pltpu.sync_copy(s_innerF1.at[sl], t_g2)

        def f1b(i, _):
            t_g3[pl.ds(i * 16, 16)] = (t_g1[pl.ds(i * 16, 16)]
                                       + t_g2[pl.ds(i * 16, 16)])
            return 0

        lax.fori_loop(0, VR, f1b, 0)
        pltpu.sync_copy(t_g3, f2w1_out.at[sl])

    wid = cid * 16 + sid
    rows_per_w = N // 32
    pltpu.sync_copy(s_P.at[pl.ds(wid * rows_per_w, rows_per_w)], t_idx)

    def fb(p, _):
        pltpu.sync_copy(feat_hbm.at[t_idx.at[pl.ds(p * 128, 128)]], t_feat)
        pltpu.sync_copy(t_feat,
                        feat_out.at[pl.ds(wid * rows_per_w + p * 128, 128)])
        return 0

    lax.fori_loop(0, rows_per_w // 128, fb, 0)


@functools.partial(jax.jit, static_argnames=())
def kernel(voxel_feat, coors):
    coors = coors.astype(jnp.int32)
    mesh = plsc.VectorSubcoreMesh(core_axis_name="c", subcore_axis_name="s",
                                  num_cores=2, num_subcores=16)
    f = pl.kernel(
        _body,
        out_type=(
            jax.ShapeDtypeStruct((N, D), jnp.float32),
            jax.ShapeDtypeStruct((N * 4,), jnp.int32),
            jax.ShapeDtypeStruct((N,), jnp.int32),
            jax.ShapeDtypeStruct((N,), jnp.int32),
            jax.ShapeDtypeStruct((N,), jnp.int32),
            jax.ShapeDtypeStruct((N,), jnp.int32),
            jax.ShapeDtypeStruct((N,), jnp.int32),
            jax.ShapeDtypeStruct((N,), jnp.int32),
        ),
        mesh=mesh,
        scratch_types=[
            pltpu.VMEM((CH * 4,), jnp.int32),
            pltpu.VMEM((CH,), jnp.int32),
            pltpu.VMEM((CH,), jnp.int32),
            pltpu.VMEM((CH,), jnp.int32),
            pltpu.VMEM((CH,), jnp.int32),
            pltpu.VMEM((CH,), jnp.int32),
            pltpu.VMEM((CH,), jnp.int32),
            pltpu.VMEM((CH,), jnp.int32),
            pltpu.VMEM((W,), jnp.int32),
            pltpu.VMEM((256,), jnp.int32),
            pltpu.VMEM((N // 32,), jnp.int32),
            pltpu.VMEM((128, D), jnp.float32),
            pltpu.VMEM_SHARED((N * 4,), jnp.int32),
            pltpu.VMEM_SHARED((N,), jnp.int32),
            pltpu.VMEM_SHARED((N,), jnp.int32),
            pltpu.VMEM_SHARED((N,), jnp.int32),
            pltpu.VMEM_SHARED((N,), jnp.int32),
            pltpu.VMEM_SHARED((N,), jnp.int32),
            pltpu.VMEM_SHARED((16 * W,), jnp.int32),
            pltpu.VMEM_SHARED((W,), jnp.int32),
            pltpu.VMEM_SHARED((W,), jnp.int32),
            pltpu.VMEM_SHARED((W,), jnp.int32),
            pltpu.VMEM_SHARED((W,), jnp.int32),
            pltpu.VMEM_SHARED((256,), jnp.int32),
        ],
        compiler_params=pltpu.CompilerParams(needs_layout_passes=False),
    )
    out = f(jnp.reshape(coors, (N * 4,)), voxel_feat)
    (feat_f, coors_flat, bwi0_f, bwi1_f, l0_f, l1_f, f2w0_f, f2w1_f) = out
    return (feat_f, jnp.reshape(coors_flat, (N, 4)), bwi0_f, bwi1_f,
            l0_f, l1_f, f2w0_f, f2w1_f)


# speedup vs baseline: 17.3679x; 1.0611x over previous
import functools

import jax
import jax.numpy as jnp
from jax import lax
from jax.experimental import pallas as pl
from jax.experimental.pallas import tpu as pltpu
from jax.experimental.pallas import tpu_sc as plsc

N = 32768
D = 128
W = 25600
CH = 2048
VR = CH // 16
WSL = W // 16

MW_PER_SAMPLE = 1600
MWY = 40

FROWS0 = 768
FROWS1 = 1280
FPIECE = 32


def _iota16():
    return lax.iota(jnp.int32, 16)


def _lvl_target(cntw):
    return jnp.where(cntw < 30, 30, jnp.where(cntw < 60, 60, 100))


def _body(coors_hbm, feat_hbm,
          feat_out, coors_out, bwi0_out, bwi1_out, l0_out, l1_out,
          f2w0_out, f2w1_out,
          t_c4, t_bwi0, t_bwi1, t_g1, t_g2, t_g3, t_g4, t_g5,
          t_hist, t_hist2, t_small, t_idx, t_feat0, t_feat1,
          sem_a, sem_b, sem_c, sem_d,
          s_bwi0, s_bwi1, s_perm1, s_P, s_innerF1,
          s_hist, s_cnt0, s_cnt1, s_base0, s_base1, s_small, s_small2):
    sid = lax.axis_index("s")
    cid = lax.axis_index("c")
    base = sid * CH
    sl = pl.ds(base, CH)

    def zero_hist_full():
        z = jnp.zeros((16,), jnp.int32)

        def zb(j, _):
            t_hist[pl.ds(j * 16, 16)] = z
            return 0

        lax.fori_loop(0, W // 16, zb, 0, unroll=8)

    def zero_hist_touched():
        z = jnp.zeros((16,), jnp.int32)

        def zb(i, _):
            w = t_bwi0[pl.ds(i * 16, 16)]
            plsc.store_scatter(t_hist, [w], z)
            return 0

        lax.fori_loop(0, VR, zb, 0, unroll=4)

    def rank_local():
        def rb(i, _):
            w = t_bwi0[pl.ds(i * 16, 16)]
            cnt, last = plsc.scan_count(w)
            old = plsc.load_gather(t_hist, [w])
            t_g1[pl.ds(i * 16, 16)] = old + cnt - 1
            plsc.store_scatter(t_hist, [w], old + cnt, mask=last)
            return 0

        lax.fori_loop(0, VR, rb, 0, unroll=2)

    def combine_hist(s_cnt_dst):
        pltpu.sync_copy(t_hist, s_hist.at[pl.ds(sid * W, W)])
        plsc.subcore_barrier()
        descs = [
            pltpu.async_copy(s_hist.at[pl.ds(r * W + sid * WSL, WSL)],
                             t_hist2.at[pl.ds(r * WSL, WSL)], sem_a)
            for r in range(16)
        ]
        for d in descs:
            d.wait()

        def jb(j, _):
            acc = jnp.zeros((16,), jnp.int32)
            for r in range(16):
                o = r * WSL + j * 16
                v = t_hist2[pl.ds(o, 16)]
                t_hist2[pl.ds(o, 16)] = acc
                acc = acc + v
            t_g2[pl.ds(j * 16, 16)] = acc
            return 0

        lax.fori_loop(0, WSL // 16, jb, 0, unroll=2)
        descs = [
            pltpu.async_copy(t_hist2.at[pl.ds(r * WSL, WSL)],
                             s_hist.at[pl.ds(r * W + sid * WSL, WSL)], sem_a)
            for r in range(16)
        ]
        descs.append(pltpu.async_copy(t_g2.at[pl.ds(0, WSL)],
                                      s_cnt_dst.at[pl.ds(sid * WSL, WSL)],
                                      sem_b))
        for d in descs:
            d.wait()
        plsc.subcore_barrier()
        pltpu.sync_copy(s_hist.at[pl.ds(sid * W, W)], t_hist2)

        def fb(i, _):
            w = t_bwi0[pl.ds(i * 16, 16)]
            t_g5[pl.ds(i * 16, 16)] = (t_g1[pl.ds(i * 16, 16)]
                                       + plsc.load_gather(t_hist2, [w]))
            return 0

        lax.fori_loop(0, VR, fb, 0, unroll=4)

    def keep_scatter(s_cnt_src, fills):
        pltpu.sync_copy(s_cnt_src.at[t_bwi0], t_g2)

        def ib(i, nknd):
            nk, nd = nknd
            cntw = t_g2[pl.ds(i * 16, 16)]
            keep = t_g5[pl.ds(i * 16, 16)] < _lvl_target(cntw)
            k = jnp.where(keep, 1, 0)
            cs = plsc.cumsum(k)
            kept_rank = nk + cs - k
            drop_rank = nd + (_iota16() + 1 - cs) - (1 - k)
            t_g3[pl.ds(i * 16, 16)] = jnp.where(keep, kept_rank, drop_rank)
            t_g4[pl.ds(i * 16, 16)] = k
            s = jnp.sum(k)
            return (nk + s, nd + 16 - s)

        nk, nd = lax.fori_loop(0, VR, ib, (jnp.int32(0), jnp.int32(0)),
                               unroll=4)
        io = _iota16()
        t_small[pl.ds(0, 16)] = jnp.where(io == 0, nk,
                                          jnp.where(io == 1, nd, 0))
        pltpu.sync_copy(t_small.at[pl.ds(0, 16)],
                        s_small.at[pl.ds(sid * 16, 16)])
        plsc.subcore_barrier()
        pltpu.sync_copy(s_small, t_small)
        nk_v = plsc.load_gather(t_small, [io * 16])
        nd_v = plsc.load_gather(t_small, [io * 16 + 1])
        before = io < sid
        k_off = jnp.sum(jnp.where(before, nk_v, 0))
        d_off = jnp.sum(jnp.where(before, nd_v, 0))
        nkeep = jnp.sum(nk_v)

        def pb(i, _):
            k = t_g4[pl.ds(i * 16, 16)]
            t_g3[pl.ds(i * 16, 16)] = (t_g3[pl.ds(i * 16, 16)]
                                       + jnp.where(k > 0, k_off,
                                                   nkeep + d_off))
            return 0

        lax.fori_loop(0, VR, pb, 0, unroll=4)
        for fill, vref, dst in fills:
            if fill is not None:
                fill(vref)
            pltpu.sync_copy(vref, dst.at[t_g3])

    def base_tables():
        csl = pl.ds(sid * WSL, WSL)
        pltpu.sync_copy(s_cnt0.at[csl], t_g1.at[pl.ds(0, WSL)])
        pltpu.sync_copy(s_cnt1.at[csl], t_g4.at[pl.ds(0, WSL)])

        def counts_of(tref):
            def cb(j, runs):
                r0, r1, r2 = runs
                cnt = tref[pl.ds(j * 16, 16)]
                i0 = jnp.where((cnt > 0) & (cnt < 30), 1, 0)
                i1 = jnp.where((cnt >= 30) & (cnt < 60), 1, 0)
                i2 = jnp.where(cnt >= 60, 1, 0)
                return (r0 + jnp.sum(i0), r1 + jnp.sum(i1), r2 + jnp.sum(i2))

            return lax.fori_loop(
                0, WSL // 16, cb,
                (jnp.int32(0), jnp.int32(0), jnp.int32(0)), unroll=4)

        io = _iota16()
        a0, a1, a2 = counts_of(t_g1)
        t_small[pl.ds(0, 16)] = jnp.where(
            io == 0, a0, jnp.where(io == 1, a1, jnp.where(io == 2, a2, 0)))
        pltpu.sync_copy(t_small.at[pl.ds(0, 16)],
                        s_small.at[pl.ds(sid * 16, 16)])
        b0, b1, b2 = counts_of(t_g4)
        t_small[pl.ds(16, 16)] = jnp.where(
            io == 0, b0, jnp.where(io == 1, b1, jnp.where(io == 2, b2, 0)))
        pltpu.sync_copy(t_small.at[pl.ds(16, 16)],
                        s_small2.at[pl.ds(sid * 16, 16)])
        plsc.subcore_barrier()

        def offs(s_small_src):
            pltpu.sync_copy(s_small_src, t_small)
            c0v = plsc.load_gather(t_small, [io * 16])
            c1v = plsc.load_gather(t_small, [io * 16 + 1])
            c2v = plsc.load_gather(t_small, [io * 16 + 2])
            before = io < sid
            return (jnp.sum(jnp.where(before, c0v, 0)),
                    jnp.sum(jnp.where(before, c1v, 0)),
                    jnp.sum(jnp.where(before, c2v, 0)))

        def emit(tref, runs, s_base_dst):
            def bb(j, runs):
                r0, r1, r2 = runs
                cnt = tref[pl.ds(j * 16, 16)]
                m0 = (cnt > 0) & (cnt < 30)
                m1 = (cnt >= 30) & (cnt < 60)
                m2 = cnt >= 60
                i0 = jnp.where(m0, 1, 0)
                i1 = jnp.where(m1, 1, 0)
                i2 = jnp.where(m2, 1, 0)
                c0 = plsc.cumsum(i0)
                c1 = plsc.cumsum(i1)
                c2 = plsc.cumsum(i2)
                b = jnp.where(
                    m0, (r0 + c0 - i0) * 30,
                    jnp.where(m1, (r1 + c1 - i1) * 60,
                              (r2 + c2 - i2) * 100))
                t_g2[pl.ds(j * 16, 16)] = b
                return (r0 + jnp.sum(i0), r1 + jnp.sum(i1), r2 + jnp.sum(i2))

            lax.fori_loop(0, WSL // 16, bb, runs, unroll=2)
            pltpu.sync_copy(t_g2.at[pl.ds(0, WSL)], s_base_dst.at[csl])

        emit(t_g1, offs(s_small), s_base0)
        emit(t_g4, offs(s_small2), s_base1)

    pltpu.sync_copy(coors_hbm.at[pl.ds(base * 4, CH * 4)], t_c4)

    def a_ib(i, _):
        lanes = _iota16() * 4 + i * 64
        b = plsc.load_gather(t_c4, [lanes])
        y = plsc.load_gather(t_c4, [lanes + 2])
        x = plsc.load_gather(t_c4, [lanes + 3])
        t_bwi0[pl.ds(i * 16, 16)] = (b * MW_PER_SAMPLE
                                     + (x // 12) * MWY + (y // 12))
        t_bwi1[pl.ds(i * 16, 16)] = (b * MW_PER_SAMPLE
                                     + ((x + 6) // 12) * MWY
                                     + ((y + 6) // 12))
        return 0

    lax.fori_loop(0, VR, a_ib, 0, unroll=4)
    pltpu.sync_copy(t_bwi0, s_bwi0.at[sl])
    pltpu.sync_copy(t_bwi1, s_bwi1.at[sl])

    zero_hist_full()
    rank_local()
    combine_hist(s_cnt0)

    def fill_iota(vref):
        def ib(i, _):
            vref[pl.ds(i * 16, 16)] = _iota16() + (base + i * 16)
            return 0
        lax.fori_loop(0, VR, ib, 0, unroll=4)

    keep_scatter(s_cnt0, [(fill_iota, t_g1, s_perm1)])
    plsc.subcore_barrier()

    zero_hist_touched()
    pltpu.sync_copy(s_perm1.at[sl], t_g4)
    pltpu.sync_copy(s_bwi1.at[t_g4], t_bwi0)
    rank_local()
    combine_hist(s_cnt1)

    def fill_perm1(vref):
        pltpu.sync_copy(s_perm1.at[sl], vref)

    keep_scatter(s_cnt1, [(fill_perm1, t_g1, s_P),
                          (None, t_g5, s_innerF1)])
    plsc.subcore_barrier()

    zero_hist_touched()
    pltpu.sync_copy(s_P.at[sl], t_g4)
    pltpu.sync_copy(s_bwi0.at[t_g4], t_bwi0)
    rank_local()
    combine_hist(s_cnt0)

    base_tables()
    plsc.subcore_barrier()

    @pl.when(cid == 0)
    def _int_outputs():
        pltpu.sync_copy(s_P.at[sl], t_g4)
        pltpu.sync_copy(s_bwi1.at[t_g4], t_bwi1)
        pltpu.sync_copy(t_bwi0, bwi0_out.at[sl])
        pltpu.sync_copy(t_bwi1, bwi1_out.at[sl])
        for q in range(4):
            def cib(i, _, q=q):
                io = _iota16()
                pv = plsc.load_gather(t_g4, [io // 4 + i * 4 + q * 512])
                t_g1[pl.ds(i * 16, 16)] = pv * 4 + (io & 3)
                return 0

            lax.fori_loop(0, VR, cib, 0, unroll=4)
            pltpu.sync_copy(coors_hbm.at[t_g1],
                            t_c4.at[pl.ds(q * CH, CH)])
        pltpu.sync_copy(t_c4, coors_out.at[pl.ds(base * 4, CH * 4)])

        def lvl_out(s_cnt_src, bwi_ref, dst):
            pltpu.sync_copy(s_cnt_src.at[bwi_ref], t_g1)

            def ib(i, _):
                cnt = t_g1[pl.ds(i * 16, 16)]
                t_g2[pl.ds(i * 16, 16)] = jnp.where(
                    cnt < 30, 0, jnp.where(cnt < 60, 1, 2))
                return 0

            lax.fori_loop(0, VR, ib, 0, unroll=4)
            pltpu.sync_copy(t_g2, dst.at[sl])

        lvl_out(s_cnt0, t_bwi0, l0_out)
        lvl_out(s_cnt1, t_bwi1, l1_out)

        pltpu.sync_copy(s_base0.at[t_bwi0], t_g1)

        def f0b(i, _):
            t_g3[pl.ds(i * 16, 16)] = (t_g1[pl.ds(i * 16, 16)]
                                       + t_g5[pl.ds(i * 16, 16)])
            return 0

        lax.fori_loop(0, VR, f0b, 0, unroll=4)
        pltpu.sync_copy(t_g3, f2w0_out.at[sl])

        pltpu.sync_copy(s_base1.at[t_bwi1], t_g1)
        pltpu.sync_copy(s_innerF1.at[sl], t_g2)

        def f1b(i, _):
            t_g3[pl.ds(i * 16, 16)] = (t_g1[pl.ds(i * 16, 16)]
                                       + t_g2[pl.ds(i * 16, 16)])
            return 0

        lax.fori_loop(0, VR, f1b, 0, unroll=4)
        pltpu.sync_copy(t_g3, f2w1_out.at[sl])

    frows = jnp.where(cid == 0, FROWS0, FROWS1)
    fstart = jnp.where(cid == 0, sid * FROWS0, 16 * FROWS0 + sid * FROWS1)
    pltpu.sync_copy(s_P.at[pl.ds(fstart, FROWS1)], t_idx)
    npieces = frows // FPIECE

    bufs = (t_feat0, t_feat1)
    gsems = (sem_a, sem_b)
    wsems = (sem_c, sem_d)

    def gather_piece(p, b):
        return pltpu.async_copy(
            feat_hbm.at[t_idx.at[pl.ds(p * FPIECE, FPIECE)]],
            bufs[b], gsems[b])

    def write_piece(p, b):
        return pltpu.async_copy(
            bufs[b], feat_out.at[pl.ds(fstart + p * FPIECE, FPIECE)],
            wsems[b])

    MAXP = FROWS1 // FPIECE
    gds = {0: gather_piece(0, 0)}
    for p in range(MAXP):
        b = p & 1
        if p + 1 < MAXP:
            @pl.when(p + 1 < npieces)
            def _(p=p):
                gds[p + 1] = gather_piece(p + 1, (p + 1) & 1)
        @pl.when(p < npieces)
        def _(p=p, b=b):
            gds[p].wait()
            write_piece(p, b).wait()


@functools.partial(jax.jit, static_argnames=())
def kernel(voxel_feat, coors):
    coors = coors.astype(jnp.int32)
    mesh = plsc.VectorSubcoreMesh(core_axis_name="c", subcore_axis_name="s",
                                  num_cores=2, num_subcores=16)
    f = pl.kernel(
        _body,
        out_type=(
            jax.ShapeDtypeStruct((N, D), jnp.float32),
            jax.ShapeDtypeStruct((N * 4,), jnp.int32),
            jax.ShapeDtypeStruct((N,), jnp.int32),
            jax.ShapeDtypeStruct((N,), jnp.int32),
            jax.ShapeDtypeStruct((N,), jnp.int32),
            jax.ShapeDtypeStruct((N,), jnp.int32),
            jax.ShapeDtypeStruct((N,), jnp.int32),
            jax.ShapeDtypeStruct((N,), jnp.int32),
        ),
        mesh=mesh,
        scratch_types=[
            pltpu.VMEM((CH * 4,), jnp.int32),
            pltpu.VMEM((CH,), jnp.int32),
            pltpu.VMEM((CH,), jnp.int32),
            pltpu.VMEM((CH,), jnp.int32),
            pltpu.VMEM((CH,), jnp.int32),
            pltpu.VMEM((CH,), jnp.int32),
            pltpu.VMEM((CH,), jnp.int32),
            pltpu.VMEM((CH,), jnp.int32),
            pltpu.VMEM((W,), jnp.int32),
            pltpu.VMEM((W,), jnp.int32),
            pltpu.VMEM((256,), jnp.int32),
            pltpu.VMEM((FROWS1,), jnp.int32),
            pltpu.VMEM((FPIECE, D), jnp.float32),
            pltpu.VMEM((FPIECE, D), jnp.float32),
            pltpu.SemaphoreType.DMA,
            pltpu.SemaphoreType.DMA,
            pltpu.SemaphoreType.DMA,
            pltpu.SemaphoreType.DMA,
            pltpu.VMEM_SHARED((N,), jnp.int32),
            pltpu.VMEM_SHARED((N,), jnp.int32),
            pltpu.VMEM_SHARED((N,), jnp.int32),
            pltpu.VMEM_SHARED((N,), jnp.int32),
            pltpu.VMEM_SHARED((N,), jnp.int32),
            pltpu.VMEM_SHARED((16 * W,), jnp.int32),
            pltpu.VMEM_SHARED((W,), jnp.int32),
            pltpu.VMEM_SHARED((W,), jnp.int32),
            pltpu.VMEM_SHARED((W,), jnp.int32),
            pltpu.VMEM_SHARED((W,), jnp.int32),
            pltpu.VMEM_SHARED((256,), jnp.int32),
            pltpu.VMEM_SHARED((256,), jnp.int32),
        ],
        compiler_params=pltpu.CompilerParams(needs_layout_passes=False),
    )
    out = f(jnp.reshape(coors, (N * 4,)), voxel_feat)
    (feat_f, coors_flat, bwi0_f, bwi1_f, l0_f, l1_f, f2w0_f, f2w1_f) = out
    return (feat_f, jnp.reshape(coors_flat, (N, 4)), bwi0_f, bwi1_f,
            l0_f, l1_f, f2w0_f, f2w1_f)


# speedup vs baseline: 18.4462x; 1.0621x over previous
import functools

import jax
import jax.numpy as jnp
from jax import lax
from jax.experimental import pallas as pl
from jax.experimental.pallas import tpu as pltpu
from jax.experimental.pallas import tpu_sc as plsc

N = 32768
D = 128
W = 25600
CH = 2048
VR = CH // 16
WSL = W // 16

MW_PER_SAMPLE = 1600
MWY = 40

FROWS0 = 256
FROWS1 = 1792
FPIECE = 32


def _iota16():
    return lax.iota(jnp.int32, 16)


def _lvl_target(cntw):
    return jnp.where(cntw < 30, 30, jnp.where(cntw < 60, 60, 100))


def _body(coors_hbm, feat_hbm,
          feat_out, coors_out, bwi0_out, bwi1_out, l0_out, l1_out,
          f2w0_out, f2w1_out,
          t_c4, t_bwi0, t_bwi1, t_g1, t_g2, t_g3, t_g4, t_g5,
          t_hist, t_hist2, t_small, t_idx, t_feat0, t_feat1,
          sem_a, sem_b, sem_c, sem_d,
          s_bwi0, s_bwi1, s_perm1, s_P, s_innerF1,
          s_hist, s_cnt0, s_cnt1, s_base0, s_base1, s_small, s_small2):
    sid = lax.axis_index("s")
    cid = lax.axis_index("c")
    base = sid * CH
    sl = pl.ds(base, CH)

    def zero_hist_full():
        z = jnp.zeros((16,), jnp.int32)

        def zb(j, _):
            t_hist[pl.ds(j * 16, 16)] = z
            return 0

        lax.fori_loop(0, W // 16, zb, 0, unroll=8)

    def zero_hist_touched():
        z = jnp.zeros((16,), jnp.int32)

        def zb(i, _):
            w = t_bwi0[pl.ds(i * 16, 16)]
            plsc.store_scatter(t_hist, [w], z)
            return 0

        lax.fori_loop(0, VR, zb, 0, unroll=4)

    def rank_local():
        def rb(i, _):
            w = t_bwi0[pl.ds(i * 16, 16)]
            cnt, last = plsc.scan_count(w)
            old = plsc.load_gather(t_hist, [w])
            t_g1[pl.ds(i * 16, 16)] = old + cnt - 1
            plsc.store_scatter(t_hist, [w], old + cnt, mask=last)
            return 0

        lax.fori_loop(0, VR, rb, 0, unroll=2)

    def combine_hist(s_cnt_dst):
        pltpu.sync_copy(t_hist, s_hist.at[pl.ds(sid * W, W)])
        plsc.subcore_barrier()
        descs = [
            pltpu.async_copy(s_hist.at[pl.ds(r * W + sid * WSL, WSL)],
                             t_hist2.at[pl.ds(r * WSL, WSL)], sem_a)
            for r in range(16)
        ]
        for d in descs:
            d.wait()

        def jb(j, _):
            acc = jnp.zeros((16,), jnp.int32)
            for r in range(16):
                o = r * WSL + j * 16
                v = t_hist2[pl.ds(o, 16)]
                t_hist2[pl.ds(o, 16)] = acc
                acc = acc + v
            t_g2[pl.ds(j * 16, 16)] = acc
            return 0

        lax.fori_loop(0, WSL // 16, jb, 0, unroll=4)
        descs = [
            pltpu.async_copy(t_hist2.at[pl.ds(r * WSL, WSL)],
                             s_hist.at[pl.ds(r * W + sid * WSL, WSL)], sem_a)
            for r in range(16)
        ]
        descs.append(pltpu.async_copy(t_g2.at[pl.ds(0, WSL)],
                                      s_cnt_dst.at[pl.ds(sid * WSL, WSL)],
                                      sem_b))
        for d in descs:
            d.wait()
        plsc.subcore_barrier()
        pltpu.sync_copy(s_hist.at[pl.ds(sid * W, W)], t_hist2)

        def fb(i, _):
            w = t_bwi0[pl.ds(i * 16, 16)]
            t_g5[pl.ds(i * 16, 16)] = (t_g1[pl.ds(i * 16, 16)]
                                       + plsc.load_gather(t_hist2, [w]))
            return 0

        lax.fori_loop(0, VR, fb, 0, unroll=4)

    def keep_scatter(s_cnt_src, fills):
        pltpu.sync_copy(s_cnt_src.at[t_bwi0], t_g2)

        def ib(i, nknd):
            nk, nd = nknd
            cntw = t_g2[pl.ds(i * 16, 16)]
            keep = t_g5[pl.ds(i * 16, 16)] < _lvl_target(cntw)
            k = jnp.where(keep, 1, 0)
            cs = plsc.cumsum(k)
            kept_rank = nk + cs - k
            drop_rank = nd + (_iota16() + 1 - cs) - (1 - k)
            t_g3[pl.ds(i * 16, 16)] = jnp.where(keep, kept_rank, drop_rank)
            t_g4[pl.ds(i * 16, 16)] = k
            s = jnp.sum(k)
            return (nk + s, nd + 16 - s)

        nk, nd = lax.fori_loop(0, VR, ib, (jnp.int32(0), jnp.int32(0)),
                               unroll=4)
        io = _iota16()
        t_small[pl.ds(0, 16)] = jnp.where(io == 0, nk,
                                          jnp.where(io == 1, nd, 0))
        pltpu.sync_copy(t_small.at[pl.ds(0, 16)],
                        s_small.at[pl.ds(sid * 16, 16)])
        plsc.subcore_barrier()
        pltpu.sync_copy(s_small, t_small)
        nk_v = plsc.load_gather(t_small, [io * 16])
        nd_v = plsc.load_gather(t_small, [io * 16 + 1])
        before = io < sid
        k_off = jnp.sum(jnp.where(before, nk_v, 0))
        d_off = jnp.sum(jnp.where(before, nd_v, 0))
        nkeep = jnp.sum(nk_v)

        def pb(i, _):
            k = t_g4[pl.ds(i * 16, 16)]
            t_g3[pl.ds(i * 16, 16)] = (t_g3[pl.ds(i * 16, 16)]
                                       + jnp.where(k > 0, k_off,
                                                   nkeep + d_off))
            return 0

        lax.fori_loop(0, VR, pb, 0, unroll=4)
        for fill, vref, dst in fills:
            if fill is not None:
                fill(vref)
            pltpu.sync_copy(vref, dst.at[t_g3])

    def base_tables():
        csl = pl.ds(sid * WSL, WSL)
        pltpu.sync_copy(s_cnt0.at[csl], t_g1.at[pl.ds(0, WSL)])
        pltpu.sync_copy(s_cnt1.at[csl], t_g4.at[pl.ds(0, WSL)])

        def counts_of(tref):
            def cb(j, runs):
                r0, r1, r2 = runs
                cnt = tref[pl.ds(j * 16, 16)]
                i0 = jnp.where((cnt > 0) & (cnt < 30), 1, 0)
                i1 = jnp.where((cnt >= 30) & (cnt < 60), 1, 0)
                i2 = jnp.where(cnt >= 60, 1, 0)
                return (r0 + jnp.sum(i0), r1 + jnp.sum(i1), r2 + jnp.sum(i2))

            return lax.fori_loop(
                0, WSL // 16, cb,
                (jnp.int32(0), jnp.int32(0), jnp.int32(0)), unroll=4)

        io = _iota16()
        a0, a1, a2 = counts_of(t_g1)
        t_small[pl.ds(0, 16)] = jnp.where(
            io == 0, a0, jnp.where(io == 1, a1, jnp.where(io == 2, a2, 0)))
        pltpu.sync_copy(t_small.at[pl.ds(0, 16)],
                        s_small.at[pl.ds(sid * 16, 16)])
        b0, b1, b2 = counts_of(t_g4)
        t_small[pl.ds(16, 16)] = jnp.where(
            io == 0, b0, jnp.where(io == 1, b1, jnp.where(io == 2, b2, 0)))
        pltpu.sync_copy(t_small.at[pl.ds(16, 16)],
                        s_small2.at[pl.ds(sid * 16, 16)])
        plsc.subcore_barrier()

        def offs(s_small_src):
            pltpu.sync_copy(s_small_src, t_small)
            c0v = plsc.load_gather(t_small, [io * 16])
            c1v = plsc.load_gather(t_small, [io * 16 + 1])
            c2v = plsc.load_gather(t_small, [io * 16 + 2])
            before = io < sid
            return (jnp.sum(jnp.where(before, c0v, 0)),
                    jnp.sum(jnp.where(before, c1v, 0)),
                    jnp.sum(jnp.where(before, c2v, 0)))

        def emit(tref, runs, s_base_dst):
            def bb(j, runs):
                r0, r1, r2 = runs
                cnt = tref[pl.ds(j * 16, 16)]
                m0 = (cnt > 0) & (cnt < 30)
                m1 = (cnt >= 30) & (cnt < 60)
                m2 = cnt >= 60
                i0 = jnp.where(m0, 1, 0)
                i1 = jnp.where(m1, 1, 0)
                i2 = jnp.where(m2, 1, 0)
                c0 = plsc.cumsum(i0)
                c1 = plsc.cumsum(i1)
                c2 = plsc.cumsum(i2)
                b = jnp.where(
                    m0, (r0 + c0 - i0) * 30,
                    jnp.where(m1, (r1 + c1 - i1) * 60,
                              (r2 + c2 - i2) * 100))
                t_g2[pl.ds(j * 16, 16)] = b
                return (r0 + jnp.sum(i0), r1 + jnp.sum(i1), r2 + jnp.sum(i2))

            lax.fori_loop(0, WSL // 16, bb, runs, unroll=2)
            pltpu.sync_copy(t_g2.at[pl.ds(0, WSL)], s_base_dst.at[csl])

        emit(t_g1, offs(s_small), s_base0)
        emit(t_g4, offs(s_small2), s_base1)

    pltpu.sync_copy(coors_hbm.at[pl.ds(base * 4, CH * 4)], t_c4)

    def a_ib(i, _):
        lanes = _iota16() * 4 + i * 64
        b = plsc.load_gather(t_c4, [lanes])
        y = plsc.load_gather(t_c4, [lanes + 2])
        x = plsc.load_gather(t_c4, [lanes + 3])
        t_bwi0[pl.ds(i * 16, 16)] = (b * MW_PER_SAMPLE
                                     + (x // 12) * MWY + (y // 12))
        t_bwi1[pl.ds(i * 16, 16)] = (b * MW_PER_SAMPLE
                                     + ((x + 6) // 12) * MWY
                                     + ((y + 6) // 12))
        return 0

    lax.fori_loop(0, VR, a_ib, 0, unroll=4)
    pltpu.sync_copy(t_bwi0, s_bwi0.at[sl])
    pltpu.sync_copy(t_bwi1, s_bwi1.at[sl])

    zero_hist_full()
    rank_local()
    combine_hist(s_cnt0)

    def fill_iota(vref):
        def ib(i, _):
            vref[pl.ds(i * 16, 16)] = _iota16() + (base + i * 16)
            return 0
        lax.fori_loop(0, VR, ib, 0, unroll=4)

    keep_scatter(s_cnt0, [(fill_iota, t_g1, s_perm1)])
    plsc.subcore_barrier()

    zero_hist_touched()
    pltpu.sync_copy(s_perm1.at[sl], t_g4)
    pltpu.sync_copy(s_bwi1.at[t_g4], t_bwi0)
    rank_local()
    combine_hist(s_cnt1)

    def fill_perm1(vref):
        pltpu.sync_copy(s_perm1.at[sl], vref)

    keep_scatter(s_cnt1, [(fill_perm1, t_g1, s_P),
                          (None, t_g5, s_innerF1)])
    plsc.subcore_barrier()

    @pl.when(cid == 0)
    def _int_outputs():
        zero_hist_touched()
        pltpu.sync_copy(s_P.at[sl], t_g4)
        pltpu.sync_copy(s_bwi0.at[t_g4], t_bwi0)
        rank_local()
        combine_hist(s_cnt0)
        base_tables()
        plsc.subcore_barrier()
        pltpu.sync_copy(s_P.at[sl], t_g4)
        pltpu.sync_copy(s_bwi1.at[t_g4], t_bwi1)
        pltpu.sync_copy(t_bwi0, bwi0_out.at[sl])
        pltpu.sync_copy(t_bwi1, bwi1_out.at[sl])
        for q in range(4):
            def cib(i, _, q=q):
                io = _iota16()
                pv = plsc.load_gather(t_g4, [io // 4 + i * 4 + q * 512])
                t_g1[pl.ds(i * 16, 16)] = pv * 4 + (io & 3)
                return 0

            lax.fori_loop(0, VR, cib, 0, unroll=4)
            pltpu.sync_copy(coors_hbm.at[t_g1],
                            t_c4.at[pl.ds(q * CH, CH)])
        pltpu.sync_copy(t_c4, coors_out.at[pl.ds(base * 4, CH * 4)])

        def lvl_out(s_cnt_src, bwi_ref, dst):
            pltpu.sync_copy(s_cnt_src.at[bwi_ref], t_g1)

            def ib(i, _):
                cnt = t_g1[pl.ds(i * 16, 16)]
                t_g2[pl.ds(i * 16, 16)] = jnp.where(
                    cnt < 30, 0, jnp.where(cnt < 60, 1, 2))
                return 0

            lax.fori_loop(0, VR, ib, 0, unroll=4)
            pltpu.sync_copy(t_g2, dst.at[sl])

        lvl_out(s_cnt0, t_bwi0, l0_out)
        lvl_out(s_cnt1, t_bwi1, l1_out)

        pltpu.sync_copy(s_base0.at[t_bwi0], t_g1)

        def f0b(i, _):
            t_g3[pl.ds(i * 16, 16)] = (t_g1[pl.ds(i * 16, 16)]
                                       + t_g5[pl.ds(i * 16, 16)])
            return 0

        lax.fori_loop(0, VR, f0b, 0, unroll=4)
        pltpu.sync_copy(t_g3, f2w0_out.at[sl])

        pltpu.sync_copy(s_base1.at[t_bwi1], t_g1)
        pltpu.sync_copy(s_innerF1.at[sl], t_g2)

        def f1b(i, _):
            t_g3[pl.ds(i * 16, 16)] = (t_g1[pl.ds(i * 16, 16)]
                                       + t_g2[pl.ds(i * 16, 16)])
            return 0

        lax.fori_loop(0, VR, f1b, 0, unroll=4)
        pltpu.sync_copy(t_g3, f2w1_out.at[sl])

    frows = jnp.where(cid == 0, FROWS0, FROWS1)
    fstart = jnp.where(cid == 0, sid * FROWS0, 16 * FROWS0 + sid * FROWS1)
    fstart = pl.multiple_of(fstart, 256)
    pltpu.sync_copy(s_P.at[pl.ds(fstart, FROWS1)], t_idx)
    npieces = frows // FPIECE

    bufs = (t_feat0, t_feat1)
    gsems = (sem_a, sem_b)
    wsems = (sem_c, sem_d)

    def gather_piece(p, b):
        return pltpu.async_copy(
            feat_hbm.at[t_idx.at[pl.ds(p * FPIECE, FPIECE)]],
            bufs[b], gsems[b])

    def write_piece(p, b):
        off = pl.multiple_of(fstart + p * FPIECE, 32)
        return pltpu.async_copy(
            bufs[b], feat_out.at[pl.ds(off, FPIECE)], wsems[b])

    MAXP = FROWS1 // FPIECE
    gds = {0: gather_piece(0, 0)}
    for p in range(MAXP):
        b = p & 1
        if p + 1 < MAXP:
            @pl.when(p + 1 < npieces)
            def _(p=p):
                gds[p + 1] = gather_piece(p + 1, (p + 1) & 1)
        @pl.when(p < npieces)
        def _(p=p, b=b):
            gds[p].wait()
            write_piece(p, b).wait()


@functools.partial(jax.jit, static_argnames=())
def kernel(voxel_feat, coors):
    coors = coors.astype(jnp.int32)
    mesh = plsc.VectorSubcoreMesh(core_axis_name="c", subcore_axis_name="s",
                                  num_cores=2, num_subcores=16)
    f = pl.kernel(
        _body,
        out_type=(
            jax.ShapeDtypeStruct((N, D), jnp.float32),
            jax.ShapeDtypeStruct((N * 4,), jnp.int32),
            jax.ShapeDtypeStruct((N,), jnp.int32),
            jax.ShapeDtypeStruct((N,), jnp.int32),
            jax.ShapeDtypeStruct((N,), jnp.int32),
            jax.ShapeDtypeStruct((N,), jnp.int32),
            jax.ShapeDtypeStruct((N,), jnp.int32),
            jax.ShapeDtypeStruct((N,), jnp.int32),
        ),
        mesh=mesh,
        scratch_types=[
            pltpu.VMEM((CH * 4,), jnp.int32),
            pltpu.VMEM((CH,), jnp.int32),
            pltpu.VMEM((CH,), jnp.int32),
            pltpu.VMEM((CH,), jnp.int32),
            pltpu.VMEM((CH,), jnp.int32),
            pltpu.VMEM((CH,), jnp.int32),
            pltpu.VMEM((CH,), jnp.int32),
            pltpu.VMEM((CH,), jnp.int32),
            pltpu.VMEM((W,), jnp.int32),
            pltpu.VMEM((W,), jnp.int32),
            pltpu.VMEM((256,), jnp.int32),
            pltpu.VMEM((FROWS1,), jnp.int32),
            pltpu.VMEM((FPIECE, D), jnp.float32),
            pltpu.VMEM((FPIECE, D), jnp.float32),
            pltpu.SemaphoreType.DMA,
            pltpu.SemaphoreType.DMA,
            pltpu.SemaphoreType.DMA,
            pltpu.SemaphoreType.DMA,
            pltpu.VMEM_SHARED((N,), jnp.int32),
            pltpu.VMEM_SHARED((N,), jnp.int32),
            pltpu.VMEM_SHARED((N,), jnp.int32),
            pltpu.VMEM_SHARED((N,), jnp.int32),
            pltpu.VMEM_SHARED((N,), jnp.int32),
            pltpu.VMEM_SHARED((16 * W,), jnp.int32),
            pltpu.VMEM_SHARED((W,), jnp.int32),
            pltpu.VMEM_SHARED((W,), jnp.int32),
            pltpu.VMEM_SHARED((W,), jnp.int32),
            pltpu.VMEM_SHARED((W,), jnp.int32),
            pltpu.VMEM_SHARED((256,), jnp.int32),
            pltpu.VMEM_SHARED((256,), jnp.int32),
        ],
        compiler_params=pltpu.CompilerParams(needs_layout_passes=False),
    )
    out = f(jnp.reshape(coors, (N * 4,)), voxel_feat)
    (feat_f, coors_flat, bwi0_f, bwi1_f, l0_f, l1_f, f2w0_f, f2w1_f) = out
    return (feat_f, jnp.reshape(coors_flat, (N, 4)), bwi0_f, bwi1_f,
            l0_f, l1_f, f2w0_f, f2w1_f)


# speedup vs baseline: 18.9836x; 1.0291x over previous
import functools

import jax
import jax.numpy as jnp
from jax import lax
from jax.experimental import pallas as pl
from jax.experimental.pallas import tpu as pltpu
from jax.experimental.pallas import tpu_sc as plsc

N = 32768
D = 128
W = 25600
CH = 2048
VR = CH // 16
WSL = W // 16

MW_PER_SAMPLE = 1600
MWY = 40

FROWS0 = 64
FROWS1 = 1984
FPIECE = 32


def _iota16():
    return lax.iota(jnp.int32, 16)


def _lvl_target(cntw):
    return jnp.where(cntw < 30, 30, jnp.where(cntw < 60, 60, 100))


def _body(coors_hbm, feat_hbm,
          feat_out, coors_out, bwi0_out, bwi1_out, l0_out, l1_out,
          f2w0_out, f2w1_out,
          t_c4, t_bwi0, t_bwi1, t_g1, t_g2, t_g3, t_g4, t_g5,
          t_hist, t_hist2, t_small, t_idx, t_feat0, t_feat1,
          sem_a, sem_b, sem_c, sem_d,
          s_bwi0, s_bwi1, s_perm1, s_P, s_innerF1,
          s_hist, s_cnt0, s_cnt1, s_base0, s_base1, s_small, s_small2):
    sid = lax.axis_index("s")
    cid = lax.axis_index("c")
    base = sid * CH
    sl = pl.ds(base, CH)

    def zero_hist_full():
        z = jnp.zeros((16,), jnp.int32)

        def zb(j, _):
            t_hist[pl.ds(j * 16, 16)] = z
            return 0

        lax.fori_loop(0, W // 16, zb, 0, unroll=8)

    def zero_hist_touched():
        z = jnp.zeros((16,), jnp.int32)

        def zb(i, _):
            w = t_bwi0[pl.ds(i * 16, 16)]
            plsc.store_scatter(t_hist, [w], z)
            return 0

        lax.fori_loop(0, VR, zb, 0, unroll=4)

    def rank_local():
        def rb(i, _):
            w = t_bwi0[pl.ds(i * 16, 16)]
            cnt, last = plsc.scan_count(w)
            old = plsc.load_gather(t_hist, [w])
            t_g1[pl.ds(i * 16, 16)] = old + cnt - 1
            plsc.store_scatter(t_hist, [w], old + cnt, mask=last)
            return 0

        lax.fori_loop(0, VR, rb, 0, unroll=2)

    def combine_hist(s_cnt_dst):
        pltpu.sync_copy(t_hist, s_hist.at[pl.ds(sid * W, W)])
        plsc.subcore_barrier()
        descs = [
            pltpu.async_copy(s_hist.at[pl.ds(r * W + sid * WSL, WSL)],
                             t_hist2.at[pl.ds(r * WSL, WSL)], sem_a)
            for r in range(16)
        ]
        for d in descs:
            d.wait()

        def jb(j, _):
            acc = jnp.zeros((16,), jnp.int32)
            for r in range(16):
                o = r * WSL + j * 16
                v = t_hist2[pl.ds(o, 16)]
                t_hist2[pl.ds(o, 16)] = acc
                acc = acc + v
            t_g2[pl.ds(j * 16, 16)] = acc
            return 0

        lax.fori_loop(0, WSL // 16, jb, 0, unroll=4)
        descs = [
            pltpu.async_copy(t_hist2.at[pl.ds(r * WSL, WSL)],
                             s_hist.at[pl.ds(r * W + sid * WSL, WSL)], sem_a)
            for r in range(16)
        ]
        descs.append(pltpu.async_copy(t_g2.at[pl.ds(0, WSL)],
                                      s_cnt_dst.at[pl.ds(sid * WSL, WSL)],
                                      sem_b))
        for d in descs:
            d.wait()
        plsc.subcore_barrier()
        def xb(i, _):
            t_g3[pl.ds(i * 16, 16)] = t_bwi0[pl.ds(i * 16, 16)] + sid * W
            return 0

        lax.fori_loop(0, VR, xb, 0, unroll=4)
        pltpu.sync_copy(s_hist.at[t_g3], t_g2)

        def fb(i, _):
            t_g5[pl.ds(i * 16, 16)] = (t_g1[pl.ds(i * 16, 16)]
                                       + t_g2[pl.ds(i * 16, 16)])
            return 0

        lax.fori_loop(0, VR, fb, 0, unroll=4)

    def keep_scatter(s_cnt_src, fills):
        pltpu.sync_copy(s_cnt_src.at[t_bwi0], t_g2)

        def ib(i, nknd):
            nk, nd = nknd
            cntw = t_g2[pl.ds(i * 16, 16)]
            keep = t_g5[pl.ds(i * 16, 16)] < _lvl_target(cntw)
            k = jnp.where(keep, 1, 0)
            cs = plsc.cumsum(k)
            kept_rank = nk + cs - k
            drop_rank = nd + (_iota16() + 1 - cs) - (1 - k)
            t_g3[pl.ds(i * 16, 16)] = jnp.where(keep, kept_rank, drop_rank)
            t_g4[pl.ds(i * 16, 16)] = k
            s = jnp.sum(k)
            return (nk + s, nd + 16 - s)

        nk, nd = lax.fori_loop(0, VR, ib, (jnp.int32(0), jnp.int32(0)),
                               unroll=4)
        io = _iota16()
        t_small[pl.ds(0, 16)] = jnp.where(io == 0, nk,
                                          jnp.where(io == 1, nd, 0))
        pltpu.sync_copy(t_small.at[pl.ds(0, 16)],
                        s_small.at[pl.ds(sid * 16, 16)])
        plsc.subcore_barrier()
        pltpu.sync_copy(s_small, t_small)
        nk_v = plsc.load_gather(t_small, [io * 16])
        nd_v = plsc.load_gather(t_small, [io * 16 + 1])
        before = io < sid
        k_off = jnp.sum(jnp.where(before, nk_v, 0))
        d_off = jnp.sum(jnp.where(before, nd_v, 0))
        nkeep = jnp.sum(nk_v)

        def pb(i, _):
            k = t_g4[pl.ds(i * 16, 16)]
            t_g3[pl.ds(i * 16, 16)] = (t_g3[pl.ds(i * 16, 16)]
                                       + jnp.where(k > 0, k_off,
                                                   nkeep + d_off))
            return 0

        lax.fori_loop(0, VR, pb, 0, unroll=4)
        for fill, vref, dst in fills:
            if fill is not None:
                fill(vref)
            pltpu.sync_copy(vref, dst.at[t_g3])

    def base_tables():
        csl = pl.ds(sid * WSL, WSL)
        pltpu.sync_copy(s_cnt0.at[csl], t_g1.at[pl.ds(0, WSL)])
        pltpu.sync_copy(s_cnt1.at[csl], t_g4.at[pl.ds(0, WSL)])

        def counts_of(tref):
            def cb(j, runs):
                r0, r1, r2 = runs
                cnt = tref[pl.ds(j * 16, 16)]
                i0 = jnp.where((cnt > 0) & (cnt < 30), 1, 0)
                i1 = jnp.where((cnt >= 30) & (cnt < 60), 1, 0)
                i2 = jnp.where(cnt >= 60, 1, 0)
                return (r0 + jnp.sum(i0), r1 + jnp.sum(i1), r2 + jnp.sum(i2))

            return lax.fori_loop(
                0, WSL // 16, cb,
                (jnp.int32(0), jnp.int32(0), jnp.int32(0)), unroll=4)

        io = _iota16()
        a0, a1, a2 = counts_of(t_g1)
        t_small[pl.ds(0, 16)] = jnp.where(
            io == 0, a0, jnp.where(io == 1, a1, jnp.where(io == 2, a2, 0)))
        pltpu.sync_copy(t_small.at[pl.ds(0, 16)],
                        s_small.at[pl.ds(sid * 16, 16)])
        b0, b1, b2 = counts_of(t_g4)
        t_small[pl.ds(16, 16)] = jnp.where(
            io == 0, b0, jnp.where(io == 1, b1, jnp.where(io == 2, b2, 0)))
        pltpu.sync_copy(t_small.at[pl.ds(16, 16)],
                        s_small2.at[pl.ds(sid * 16, 16)])
        plsc.subcore_barrier()

        def offs(s_small_src):
            pltpu.sync_copy(s_small_src, t_small)
            c0v = plsc.load_gather(t_small, [io * 16])
            c1v = plsc.load_gather(t_small, [io * 16 + 1])
            c2v = plsc.load_gather(t_small, [io * 16 + 2])
            before = io < sid
            return (jnp.sum(jnp.where(before, c0v, 0)),
                    jnp.sum(jnp.where(before, c1v, 0)),
                    jnp.sum(jnp.where(before, c2v, 0)))

        def emit(tref, runs, s_base_dst):
            def bb(j, runs):
                r0, r1, r2 = runs
                cnt = tref[pl.ds(j * 16, 16)]
                m0 = (cnt > 0) & (cnt < 30)
                m1 = (cnt >= 30) & (cnt < 60)
                m2 = cnt >= 60
                i0 = jnp.where(m0, 1, 0)
                i1 = jnp.where(m1, 1, 0)
                i2 = jnp.where(m2, 1, 0)
                c0 = plsc.cumsum(i0)
                c1 = plsc.cumsum(i1)
                c2 = plsc.cumsum(i2)
                b = jnp.where(
                    m0, (r0 + c0 - i0) * 30,
                    jnp.where(m1, (r1 + c1 - i1) * 60,
                              (r2 + c2 - i2) * 100))
                t_g2[pl.ds(j * 16, 16)] = b
                return (r0 + jnp.sum(i0), r1 + jnp.sum(i1), r2 + jnp.sum(i2))

            lax.fori_loop(0, WSL // 16, bb, runs, unroll=2)
            pltpu.sync_copy(t_g2.at[pl.ds(0, WSL)], s_base_dst.at[csl])

        emit(t_g1, offs(s_small), s_base0)
        emit(t_g4, offs(s_small2), s_base1)

    pltpu.sync_copy(coors_hbm.at[pl.ds(base * 4, CH * 4)], t_c4)

    def a_ib(i, _):
        lanes = _iota16() * 4 + i * 64
        b = plsc.load_gather(t_c4, [lanes])
        y = plsc.load_gather(t_c4, [lanes + 2])
        x = plsc.load_gather(t_c4, [lanes + 3])
        t_bwi0[pl.ds(i * 16, 16)] = (b * MW_PER_SAMPLE
                                     + (x // 12) * MWY + (y // 12))
        t_bwi1[pl.ds(i * 16, 16)] = (b * MW_PER_SAMPLE
                                     + ((x + 6) // 12) * MWY
                                     + ((y + 6) // 12))
        return 0

    lax.fori_loop(0, VR, a_ib, 0, unroll=4)
    pltpu.sync_copy(t_bwi0, s_bwi0.at[sl])
    pltpu.sync_copy(t_bwi1, s_bwi1.at[sl])

    zero_hist_full()
    rank_local()
    combine_hist(s_cnt0)

    def fill_iota(vref):
        def ib(i, _):
            vref[pl.ds(i * 16, 16)] = _iota16() + (base + i * 16)
            return 0
        lax.fori_loop(0, VR, ib, 0, unroll=4)

    keep_scatter(s_cnt0, [(fill_iota, t_g1, s_perm1)])
    plsc.subcore_barrier()

    zero_hist_touched()
    pltpu.sync_copy(s_perm1.at[sl], t_g4)
    pltpu.sync_copy(s_bwi1.at[t_g4], t_bwi0)
    rank_local()
    combine_hist(s_cnt1)

    def fill_perm1(vref):
        pltpu.sync_copy(s_perm1.at[sl], vref)

    keep_scatter(s_cnt1, [(fill_perm1, t_g1, s_P),
                          (None, t_g5, s_innerF1)])
    plsc.subcore_barrier()

    bufs = (t_feat0, t_feat1)
    gsems = (sem_c, sem_d)
    wsems = (sem_a, sem_b)

    def gather_piece(fstart, p, b):
        return pltpu.async_copy(
            feat_hbm.at[t_idx.at[pl.ds(p * FPIECE, FPIECE)]],
            bufs[b], gsems[b])

    def write_piece(fstart, p, b):
        off = pl.multiple_of(fstart + p * FPIECE, 32)
        return pltpu.async_copy(
            bufs[b], feat_out.at[pl.ds(off, FPIECE)], wsems[b])

    @pl.when(cid == 0)
    def _int_outputs():
        f0 = pl.multiple_of(sid * FROWS0, 32)
        pltpu.sync_copy(s_P.at[pl.ds(f0, FROWS0)],
                        t_idx.at[pl.ds(0, FROWS0)])
        g_early = [gather_piece(f0, 0, 0), gather_piece(f0, 1, 1)]
        zero_hist_touched()
        pltpu.sync_copy(s_P.at[sl], t_g4)
        pltpu.sync_copy(s_bwi0.at[t_g4], t_bwi0)
        rank_local()
        combine_hist(s_cnt0)
        base_tables()
        plsc.subcore_barrier()
        pltpu.sync_copy(s_P.at[sl], t_g4)
        pltpu.sync_copy(s_bwi1.at[t_g4], t_bwi1)
        pltpu.sync_copy(t_bwi0, bwi0_out.at[sl])
        pltpu.sync_copy(t_bwi1, bwi1_out.at[sl])
        for q in range(4):
            def cib(i, _, q=q):
                io = _iota16()
                pv = plsc.load_gather(t_g4, [io // 4 + i * 4 + q * 512])
                t_g1[pl.ds(i * 16, 16)] = pv * 4 + (io & 3)
                return 0

            lax.fori_loop(0, VR, cib, 0, unroll=4)
            pltpu.sync_copy(coors_hbm.at[t_g1],
                            t_c4.at[pl.ds(q * CH, CH)])
        pltpu.sync_copy(t_c4, coors_out.at[pl.ds(base * 4, CH * 4)])

        def lvl_out(s_cnt_src, bwi_ref, dst):
            pltpu.sync_copy(s_cnt_src.at[bwi_ref], t_g1)

            def ib(i, _):
                cnt = t_g1[pl.ds(i * 16, 16)]
                t_g2[pl.ds(i * 16, 16)] = jnp.where(
                    cnt < 30, 0, jnp.where(cnt < 60, 1, 2))
                return 0

            lax.fori_loop(0, VR, ib, 0, unroll=4)
            pltpu.sync_copy(t_g2, dst.at[sl])

        lvl_out(s_cnt0, t_bwi0, l0_out)
        lvl_out(s_cnt1, t_bwi1, l1_out)

        pltpu.sync_copy(s_base0.at[t_bwi0], t_g1)

        def f0b(i, _):
            t_g3[pl.ds(i * 16, 16)] = (t_g1[pl.ds(i * 16, 16)]
                                       + t_g5[pl.ds(i * 16, 16)])
            return 0

        lax.fori_loop(0, VR, f0b, 0, unroll=4)
        pltpu.sync_copy(t_g3, f2w0_out.at[sl])

        pltpu.sync_copy(s_base1.at[t_bwi1], t_g1)
        pltpu.sync_copy(s_innerF1.at[sl], t_g2)

        def f1b(i, _):
            t_g3[pl.ds(i * 16, 16)] = (t_g1[pl.ds(i * 16, 16)]
                                       + t_g2[pl.ds(i * 16, 16)])
            return 0

        lax.fori_loop(0, VR, f1b, 0, unroll=4)
        pltpu.sync_copy(t_g3, f2w1_out.at[sl])

        for pp in range(2):
            g_early[pp].wait()
            write_piece(f0, pp, pp).wait()

    @pl.when(cid == 1)
    def _features1():
        f1s = pl.multiple_of(16 * FROWS0 + sid * FROWS1, 32)
        pltpu.sync_copy(s_P.at[pl.ds(f1s, FROWS1)], t_idx)
        MAXP = FROWS1 // FPIECE
        gds = {0: gather_piece(f1s, 0, 0)}
        for p in range(MAXP):
            b = p & 1
            if p + 1 < MAXP:
                gds[p + 1] = gather_piece(f1s, p + 1, (p + 1) & 1)
            gds[p].wait()
            write_piece(f1s, p, b).wait()


@functools.partial(jax.jit, static_argnames=())
def kernel(voxel_feat, coors):
    coors = coors.astype(jnp.int32)
    mesh = plsc.VectorSubcoreMesh(core_axis_name="c", subcore_axis_name="s",
                                  num_cores=2, num_subcores=16)
    f = pl.kernel(
        _body,
        out_type=(
            jax.ShapeDtypeStruct((N, D), jnp.float32),
            jax.ShapeDtypeStruct((N * 4,), jnp.int32),
            jax.ShapeDtypeStruct((N,), jnp.int32),
            jax.ShapeDtypeStruct((N,), jnp.int32),
            jax.ShapeDtypeStruct((N,), jnp.int32),
            jax.ShapeDtypeStruct((N,), jnp.int32),
            jax.ShapeDtypeStruct((N,), jnp.int32),
            jax.ShapeDtypeStruct((N,), jnp.int32),
        ),
        mesh=mesh,
        scratch_types=[
            pltpu.VMEM((CH * 4,), jnp.int32),
            pltpu.VMEM((CH,), jnp.int32),
            pltpu.VMEM((CH,), jnp.int32),
            pltpu.VMEM((CH,), jnp.int32),
            pltpu.VMEM((CH,), jnp.int32),
            pltpu.VMEM((CH,), jnp.int32),
            pltpu.VMEM((CH,), jnp.int32),
            pltpu.VMEM((CH,), jnp.int32),
            pltpu.VMEM((W,), jnp.int32),
            pltpu.VMEM((W,), jnp.int32),
            pltpu.VMEM((256,), jnp.int32),
            pltpu.VMEM((FROWS1,), jnp.int32),
            pltpu.VMEM((FPIECE, D), jnp.float32),
            pltpu.VMEM((FPIECE, D), jnp.float32),
            pltpu.SemaphoreType.DMA,
            pltpu.SemaphoreType.DMA,
            pltpu.SemaphoreType.DMA,
            pltpu.SemaphoreType.DMA,
            pltpu.VMEM_SHARED((N,), jnp.int32),
            pltpu.VMEM_SHARED((N,), jnp.int32),
            pltpu.VMEM_SHARED((N,), jnp.int32),
            pltpu.VMEM_SHARED((N,), jnp.int32),
            pltpu.VMEM_SHARED((N,), jnp.int32),
            pltpu.VMEM_SHARED((16 * W,), jnp.int32),
            pltpu.VMEM_SHARED((W,), jnp.int32),
            pltpu.VMEM_SHARED((W,), jnp.int32),
            pltpu.VMEM_SHARED((W,), jnp.int32),
            pltpu.VMEM_SHARED((W,), jnp.int32),
            pltpu.VMEM_SHARED((256,), jnp.int32),
            pltpu.VMEM_SHARED((256,), jnp.int32),
        ],
        compiler_params=pltpu.CompilerParams(needs_layout_passes=False),
    )
    out = f(jnp.reshape(coors, (N * 4,)), voxel_feat)
    (feat_f, coors_flat, bwi0_f, bwi1_f, l0_f, l1_f, f2w0_f, f2w1_f) = out
    return (feat_f, jnp.reshape(coors_flat, (N, 4)), bwi0_f, bwi1_f,
            l0_f, l1_f, f2w0_f, f2w1_f)
